# baseline (speedup 1.0000x reference)
import jax
import jax.numpy as jnp
from jax.experimental import pallas as pl
from jax.experimental.pallas import tpu as pltpu

N_USERS = 100000
N_ITEMS = 100000
DIM = 64
N_OUT = N_USERS + N_ITEMS


def _concat_tc(u_hbm, i_hbm, o_hbm, su, si):
    a = pltpu.make_async_copy(u_hbm, o_hbm.at[pl.ds(0, N_USERS), :], su)
    b = pltpu.make_async_copy(i_hbm, o_hbm.at[pl.ds(N_USERS, N_ITEMS), :], si)
    a.start()
    b.start()
    a.wait()
    b.wait()


def kernel(inputs, user_embedding, item_embedding):
    del inputs
    return pl.pallas_call(
        _concat_tc,
        out_shape=jax.ShapeDtypeStruct((N_OUT, DIM), jnp.float32),
        in_specs=[pl.BlockSpec(memory_space=pl.ANY)] * 2,
        out_specs=pl.BlockSpec(memory_space=pl.ANY),
        scratch_shapes=[pltpu.SemaphoreType.DMA] * 2,
    )(user_embedding, item_embedding)


# speedup vs baseline: 14.1668x; 14.1668x over previous
import functools

import jax
import jax.numpy as jnp
from jax import lax
from jax.experimental import pallas as pl
from jax.experimental.pallas import tpu as pltpu
from jax.experimental.pallas import tpu_sc as plsc

N_USERS = 100000
N_ITEMS = 100000
DIM = 64
N_OUT = N_USERS + N_ITEMS

NC = 2
NS = 16
CHUNK = 200
NBUF = 4
BIG_WORKERS = 13
BIG_CHUNKS = 32
SMALL_CHUNKS = 28


def _copy_shard(src_hbm, src_base, out_hbm, dst_base, bufs, isems, osems,
                nchunk):
    def in_copy(k):
        return pltpu.make_async_copy(
            src_hbm.at[pl.ds(src_base + k * CHUNK, CHUNK), :],
            bufs[k % NBUF], isems[k % NBUF])

    def out_copy(k):
        return pltpu.make_async_copy(
            bufs[k % NBUF],
            out_hbm.at[pl.ds(dst_base + k * CHUNK, CHUNK), :],
            osems[k % NBUF])

    started = min(NBUF - 1, nchunk)
    for b in range(started):
        in_copy(b).start()
    for k in range(nchunk):
        in_copy(k).wait()
        out_copy(k).start()
        nxt = k + NBUF - 1
        if nxt < nchunk and nxt >= started:
            if nxt - NBUF >= 0:
                out_copy(nxt - NBUF).wait()
            in_copy(nxt).start()
    for k in range(max(0, nchunk - NBUF), nchunk):
        out_copy(k).wait()


def _worker_shard(src_hbm, out_hbm, t, dst_off, bufs, isems, osems):
    @pl.when(t < BIG_WORKERS)
    def _():
        base = pl.multiple_of(t * (BIG_CHUNKS * CHUNK), 8)
        _copy_shard(src_hbm, base, out_hbm, dst_off + base,
                    bufs, isems, osems, BIG_CHUNKS)

    @pl.when(t >= BIG_WORKERS)
    def _():
        base = pl.multiple_of(
            BIG_WORKERS * BIG_CHUNKS * CHUNK
            + (t - BIG_WORKERS) * (SMALL_CHUNKS * CHUNK), 8)
        _copy_shard(src_hbm, base, out_hbm, dst_off + base,
                    bufs, isems, osems, SMALL_CHUNKS)


@functools.partial(
    pl.kernel,
    mesh=plsc.VectorSubcoreMesh(core_axis_name="c", subcore_axis_name="s"),
    out_type=jax.ShapeDtypeStruct((N_OUT, DIM), jnp.float32),
    scratch_types=(
        [pltpu.VMEM((CHUNK, DIM), jnp.float32)] * NBUF
        + [pltpu.SemaphoreType.DMA] * (2 * NBUF)
    ),
)
def _concat_tables(user_hbm, item_hbm, out_hbm, b0, b1, b2, b3,
                   si0, si1, si2, si3, so0, so1, so2, so3):
    bufs = (b0, b1, b2, b3)
    isems = (si0, si1, si2, si3)
    osems = (so0, so1, so2, so3)
    wid = lax.axis_index("s") * NC + lax.axis_index("c")

    @pl.when(wid < NS)
    def _():
        _worker_shard(user_hbm, out_hbm, wid, 0, bufs, isems, osems)

    @pl.when(wid >= NS)
    def _():
        _worker_shard(item_hbm, out_hbm, wid - NS, N_USERS,
                      bufs, isems, osems)


def kernel(inputs, user_embedding, item_embedding):
    del inputs
    return _concat_tables(user_embedding, item_embedding)
